# baseline (speedup 1.0000x reference)
import functools

import jax
import jax.numpy as jnp
from jax import lax
from jax.experimental import pallas as pl
from jax.experimental.pallas import tpu as pltpu
from jax.experimental.pallas import tpu_sc as plsc

_N = 10000
_E = 640000
_D = 128
_B = 64
_BERT = 768
_NC = 2
_NT = 16
_DH = _D // _NC
_NPAD = 10240
_RPT = _NPAD // _NT
_EPW = _E // (_NC * _NT)
_CH = 80
_NCH = _EPW // _CH
_NHALF = _NPAD // _NC
_RPTH = _NHALF // _NT
_ECH = _E // (_NT * _CH)
_GRP = 50
_DGRP = _NCH // _GRP
_CHA = 128
_EPAD = 655360
_ACH = _EPAD // (_NT * _CHA)
_AGRP2 = 8
_GRPA = _ACH // _AGRP2
_WB = 64
_DWB = 2048
_BLK = 1000
_F32 = jnp.float32


def _gelu(x):
    return 0.5 * x * (1.0 + lax.erf(x * 0.7071067811865476))


def _embed_body(sem_ref, sidx_ref, wsem_ref, bsem_ref, emb_ref, gamma_ref,
                beta_ref, o_ref):
    h = jnp.dot(sem_ref[...], wsem_ref[...], preferred_element_type=_F32)
    h = h + bsem_ref[...]
    si = sidx_ref[...].astype(jnp.int32)
    h = h + jnp.where(si <= 0, emb_ref[0:1, :], emb_ref[1:2, :])
    m = jnp.mean(h, axis=-1, keepdims=True)
    v = jnp.mean((h - m) * (h - m), axis=-1, keepdims=True)
    h = (h - m) * lax.rsqrt(v + 1e-5) * gamma_ref[...] + beta_ref[...]
    o_ref[...] = _gelu(h)


def _embed(sem, sidx, w_sem, b_sem, emb, gamma, beta):
    grid = (_N // _BLK,)
    return pl.pallas_call(
        _embed_body,
        grid=grid,
        in_specs=[
            pl.BlockSpec((_BLK, _BERT), lambda i: (i, 0)),
            pl.BlockSpec((_BLK, 1), lambda i: (i, 0)),
            pl.BlockSpec((_BERT, _D), lambda i: (0, 0)),
            pl.BlockSpec((1, _D), lambda i: (0, 0)),
            pl.BlockSpec((2, _D), lambda i: (0, 0)),
            pl.BlockSpec((1, _D), lambda i: (0, 0)),
            pl.BlockSpec((1, _D), lambda i: (0, 0)),
        ],
        out_specs=pl.BlockSpec((_BLK, _D), lambda i: (i, 0)),
        out_shape=jax.ShapeDtypeStruct((_N, _D), _F32),
    )(sem, sidx, w_sem, b_sem, emb, gamma, beta)


def _prep_body(h_ref, w_ref, cnt0_ref, cnt1_ref, pa_ref, pb_ref):
    g = lax.rsqrt(cnt0_ref[...] + cnt1_ref[...] + 1.0)
    p = jnp.dot(h_ref[...], w_ref[...], preferred_element_type=_F32) * g
    pa_ref[...] = p[:, :_DH]
    pb_ref[...] = p[:, _DH:]


def _prep(h, w, cnt0, cnt1):
    grid = (_N // _BLK,)
    half = pl.BlockSpec((_BLK, _DH), lambda i: (i, 0))
    return pl.pallas_call(
        _prep_body,
        grid=grid,
        in_specs=[
            pl.BlockSpec((_BLK, _D), lambda i: (i, 0)),
            pl.BlockSpec((_D, _D), lambda i: (0, 0)),
            pl.BlockSpec((_BLK, 1), lambda i: (i, 0)),
            pl.BlockSpec((_BLK, 1), lambda i: (i, 0)),
        ],
        out_specs=[half, half],
        out_shape=[
            jax.ShapeDtypeStruct((_NPAD, _DH), _F32),
            jax.ShapeDtypeStruct((_NPAD, _DH), _F32),
        ],
    )(h, w, cnt0, cnt1)


def _mid_body(agga_ref, aggb_ref, pa_ref, pb_ref, cnt0_ref, cnt1_ref,
              b1_ref, w2_ref, h1_ref, p2a_ref, p2b_ref):
    g = lax.rsqrt(cnt0_ref[...] + cnt1_ref[...] + 1.0)
    s = jnp.concatenate(
        [agga_ref[...] + pa_ref[...], aggb_ref[...] + pb_ref[...]], axis=1)
    h1 = _gelu(s * g + b1_ref[...])
    h1_ref[...] = h1
    p2 = jnp.dot(h1, w2_ref[...], preferred_element_type=_F32) * g
    p2a_ref[...] = p2[:, :_DH]
    p2b_ref[...] = p2[:, _DH:]


def _mid(agga, aggb, pa, pb, cnt0, cnt1, b1, w2):
    grid = (_N // _BLK,)
    full = pl.BlockSpec((_BLK, _D), lambda i: (i, 0))
    half = pl.BlockSpec((_BLK, _DH), lambda i: (i, 0))
    one = pl.BlockSpec((_BLK, 1), lambda i: (i, 0))
    return pl.pallas_call(
        _mid_body,
        grid=grid,
        in_specs=[
            half, half, half, half, one, one,
            pl.BlockSpec((1, _D), lambda i: (0, 0)),
            pl.BlockSpec((_D, _D), lambda i: (0, 0)),
        ],
        out_specs=[full, half, half],
        out_shape=[
            jax.ShapeDtypeStruct((_N, _D), _F32),
            jax.ShapeDtypeStruct((_NPAD, _DH), _F32),
            jax.ShapeDtypeStruct((_NPAD, _DH), _F32),
        ],
    )(agga, aggb, pa, pb, cnt0, cnt1, b1, w2)


def _final_body(h1_ref, agga_ref, aggb_ref, pa_ref, pb_ref, cnt0_ref,
                cnt1_ref, b2_ref, batch_ref, wc1_ref, bc1_ref, wc2_ref,
                bc2_ref, o_ref, sums_scr, counts_scr):
    i = pl.program_id(0)

    @pl.when(i == 0)
    def _():
        sums_scr[...] = jnp.zeros_like(sums_scr)
        counts_scr[...] = jnp.zeros_like(counts_scr)

    g = lax.rsqrt(cnt0_ref[...] + cnt1_ref[...] + 1.0)
    s = jnp.concatenate(
        [agga_ref[...] + pa_ref[...], aggb_ref[...] + pb_ref[...]], axis=1)
    h2 = h1_ref[...] + _gelu(s * g + b2_ref[...])
    onehot = (batch_ref[...] ==
              lax.broadcasted_iota(jnp.int32, (_BLK, _B), 1)).astype(_F32)
    dn = (((0,), (0,)), ((), ()))
    sums_scr[...] += lax.dot_general(onehot, h2, dn,
                                     preferred_element_type=_F32)
    counts_scr[...] += lax.dot_general(onehot, jnp.ones((_BLK, 1), _F32), dn,
                                       preferred_element_type=_F32)

    @pl.when(i == _N // _BLK - 1)
    def _():
        hg = sums_scr[...] / jnp.maximum(counts_scr[...], 1.0)
        z = _gelu(jnp.dot(hg, wc1_ref[...], preferred_element_type=_F32)
                  + bc1_ref[...])
        o_ref[...] = (jnp.dot(z, wc2_ref[...], preferred_element_type=_F32)
                      + bc2_ref[...])


def _final(h1, agga, aggb, pa, pb, cnt0, cnt1, b2, batch, wc1, bc1, wc2p,
           bc2p):
    grid = (_N // _BLK,)
    full = pl.BlockSpec((_BLK, _D), lambda i: (i, 0))
    half = pl.BlockSpec((_BLK, _DH), lambda i: (i, 0))
    one = pl.BlockSpec((_BLK, 1), lambda i: (i, 0))
    wfull = pl.BlockSpec((_D, _D), lambda i: (0, 0))
    brow = pl.BlockSpec((1, _D), lambda i: (0, 0))
    return pl.pallas_call(
        _final_body,
        grid=grid,
        in_specs=[full, half, half, half, half, one, one, brow,
                  pl.BlockSpec((_BLK, 1), lambda i: (i, 0)),
                  wfull, brow, wfull, brow],
        out_specs=pl.BlockSpec((_B, _D), lambda i: (0, 0)),
        out_shape=jax.ShapeDtypeStruct((_B, _D), _F32),
        scratch_shapes=[
            pltpu.VMEM((_B, _D), _F32),
            pltpu.VMEM((_B, 1), _F32),
        ],
        compiler_params=pltpu.CompilerParams(
            dimension_semantics=("arbitrary",)),
    )(h1, agga, aggb, pa, pb, cnt0, cnt1, b2, batch, wc1, bc1, wc2p, bc2p)


def _deg_sc(dst4):
    mesh = plsc.VectorSubcoreMesh(core_axis_name="c", subcore_axis_name="s",
                                  num_cores=_NC, num_subcores=_NT)

    @functools.partial(
        pl.kernel,
        out_type=jax.ShapeDtypeStruct((_NC, _NPAD), _F32),
        mesh=mesh,
        scratch_types=[
            pltpu.VMEM((_GRP, _CH), jnp.int32),
            pltpu.VMEM((_CH,), _F32),
            pltpu.VMEM((_DWB,), _F32),
            pltpu.VMEM_SHARED((_NPAD,), _F32),
            pltpu.SemaphoreType.DMA,
        ],
    )
    def k(dst_hbm, cnt_hbm, didx_v, ones_v, buf_v, acc_sh, dsem):
        c = lax.axis_index("c")
        s = lax.axis_index("s")
        wid = c * _NT + s

        for kk in range(_CH // 16):
            ones_v[pl.ds(kk * 16, 16)] = jnp.ones((16,), _F32)

        @pl.when(s == 0)
        def _():
            def fill_zero(i, _):
                buf_v[pl.ds(i * 16, 16)] = jnp.zeros((16,), _F32)
                return 0
            lax.fori_loop(0, _DWB // 16, fill_zero, 0)
            for t in range(_NPAD // _DWB):
                pltpu.sync_copy(buf_v, acc_sh.at[pl.ds(t * _DWB, _DWB)])

        plsc.subcore_barrier()

        def group(gi, _):
            pltpu.sync_copy(dst_hbm.at[wid, gi], didx_v)
            for sb in range(_GRP // 10):
                ds = [pltpu.async_copy(
                    ones_v, acc_sh.at[didx_v.at[sb * 10 + t]], dsem,
                    add=True) for t in range(10)]
                for d in ds:
                    d.wait()
            return 0
        lax.fori_loop(0, _DGRP, group, 0)

        plsc.subcore_barrier()

        @pl.when(s == 0)
        def _():
            for t in range(_NPAD // _DWB):
                pltpu.sync_copy(acc_sh.at[pl.ds(t * _DWB, _DWB)], buf_v)
                pltpu.sync_copy(buf_v, cnt_hbm.at[c, pl.ds(t * _DWB, _DWB)])

    return k(dst4)


def _agg_sc(pa, pb, srcA, dstA):
    mesh = plsc.VectorSubcoreMesh(core_axis_name="c", subcore_axis_name="s",
                                  num_cores=_NC, num_subcores=_NT)

    @functools.partial(
        pl.kernel,
        out_type=jax.ShapeDtypeStruct((_NC, _NPAD, _DH), _F32),
        mesh=mesh,
        scratch_types=[
            pltpu.VMEM((_GRPA, _CHA), jnp.int32),
            pltpu.VMEM((_GRPA, _CHA), jnp.int32),
            pltpu.VMEM((_CHA, _DH), _F32),
            pltpu.VMEM((_CHA, _DH), _F32),
            pltpu.VMEM((_CHA, _DH), _F32),
            pltpu.VMEM((_CHA, _DH), _F32),
            pltpu.VMEM((_CHA, _DH), _F32),
            pltpu.VMEM((_WB, _DH), _F32),
            pltpu.VMEM_SHARED((_NPAD, _DH), _F32),
            pltpu.SemaphoreType.DMA,
            pltpu.SemaphoreType.DMA,
            pltpu.SemaphoreType.DMA,
            pltpu.SemaphoreType.DMA,
            pltpu.SemaphoreType.DMA,
            pltpu.SemaphoreType.DMA,
            pltpu.SemaphoreType.DMA,
            pltpu.SemaphoreType.DMA,
            pltpu.SemaphoreType.DMA,
            pltpu.SemaphoreType.DMA,
        ],
        compiler_params=pltpu.CompilerParams(use_tc_tiling_on_sc=False),
    )
    def k(pa_hbm, pb_hbm, src_hbm, dst_hbm, out_hbm,
          sidx_v, didx_v, r0, r1, r2, r3, r4, buf_v, acc_sh,
          g0, g1, g2, g3, g4, s0, s1, s2, s3, s4):
        c = lax.axis_index("c")
        s = lax.axis_index("s")
        rows = (r0, r1, r2, r3, r4)
        gsem = (g0, g1, g2, g3, g4)
        ssem = (s0, s1, s2, s3, s4)

        def fill_zero(i, _):
            for jj in range(_DH // 16):
                buf_v[i, pl.ds(jj * 16, 16)] = jnp.zeros((16,), _F32)
            return 0
        lax.fori_loop(0, _WB, fill_zero, 0)

        for t in range(_RPT // _WB):
            pltpu.sync_copy(
                buf_v, acc_sh.at[pl.ds(s * _RPT + t * _WB, _WB)])

        plsc.subcore_barrier()

        def gath(j, b):
            @pl.when(c == 0)
            def _():
                pltpu.async_copy(pa_hbm.at[sidx_v.at[j]], rows[b], gsem[b])

            @pl.when(c == 1)
            def _():
                pltpu.async_copy(pb_hbm.at[sidx_v.at[j]], rows[b], gsem[b])

        def gwait(j, b):
            pltpu.make_async_copy(pa_hbm.at[sidx_v.at[j]], rows[b],
                                  gsem[b]).wait()

        def scat(j, b):
            pltpu.async_copy(rows[b], acc_sh.at[didx_v.at[j]],
                             ssem[b], add=True)

        def swait(j, b):
            pltpu.make_async_copy(rows[b], acc_sh.at[didx_v.at[j]],
                                  ssem[b]).wait()

        def group(gi, _):
            pltpu.sync_copy(src_hbm.at[s, gi], sidx_v)
            pltpu.sync_copy(dst_hbm.at[s, gi], didx_v)

            gath(0, 0)
            gath(1, 1)
            gwait(0, 0)
            scat(0, 0)
            gath(2, 2)
            gwait(1, 1)
            scat(1, 1)
            gath(3, 3)
            gwait(2, 2)
            scat(2, 2)
            gath(4, 4)
            gwait(3, 3)
            scat(3, 3)
            swait(0, 0)
            gath(5, 0)
            gwait(4, 4)
            scat(4, 4)
            swait(1, 1)
            gath(6, 1)

            def steady(jo, _2):
                j = jo * 5
                for b in range(5):
                    gwait(j + b, b)
                    scat(j + b, b)
                    bn = (b + 2) % 5
                    swait(j + b - 3, bn)
                    gath(j + b + 2, bn)
                return 0
            lax.fori_loop(1, _GRPA // 5 - 1, steady, 0)

            gg = _GRPA - 5
            gwait(gg, 0)
            scat(gg, 0)
            swait(gg - 3, 2)
            gath(gg + 2, 2)
            gwait(gg + 1, 1)
            scat(gg + 1, 1)
            swait(gg - 2, 3)
            gath(gg + 3, 3)
            gwait(gg + 2, 2)
            scat(gg + 2, 2)
            swait(gg - 1, 4)
            gath(gg + 4, 4)
            gwait(gg + 3, 3)
            scat(gg + 3, 3)
            gwait(gg + 4, 4)
            scat(gg + 4, 4)
            for j, b in ((gg, 0), (gg + 1, 1), (gg + 2, 2), (gg + 3, 3),
                         (gg + 4, 4)):
                swait(j, b)
            return 0
        lax.fori_loop(0, _AGRP2, group, 0)

        plsc.subcore_barrier()
        for t in range(_RPT // _WB):
            pltpu.sync_copy(
                acc_sh.at[pl.ds(s * _RPT + t * _WB, _WB)], buf_v)
            pltpu.sync_copy(
                buf_v, out_hbm.at[c, pl.ds(s * _RPT + t * _WB, _WB)])

    return k(pa, pb, srcA, dstA)


def kernel(x, edge_index, batch, W_sem, b_sem, emb, gamma, beta, W1, b1, W2,
           b2, Wc1, bc1, Wc2, bc2):
    sem_feat = x[:, :_BERT]
    sidx = x[:, _BERT:]
    src = edge_index[0]
    dst = edge_index[1]
    dst4 = dst.reshape(_NC * _NT, _DGRP, _GRP, _CH)
    epad = jnp.full((_EPAD - _E,), _NPAD - 1, jnp.int32)
    srcA = jnp.concatenate([src, epad]).reshape(_NT, _AGRP2, _GRPA, _CHA)
    dstA = jnp.concatenate([dst, epad]).reshape(_NT, _AGRP2, _GRPA, _CHA)

    cnts = _deg_sc(dst4)
    cnt0 = cnts[0].reshape(_NPAD, 1)
    cnt1 = cnts[1].reshape(_NPAD, 1)

    h0 = _embed(sem_feat, sidx, W_sem, b_sem.reshape(1, _D), emb,
                gamma.reshape(1, _D), beta.reshape(1, _D))

    p1a, p1b = _prep(h0, W1, cnt0, cnt1)
    aggs1 = _agg_sc(p1a, p1b, srcA, dstA)
    h1, p2a, p2b = _mid(aggs1[0], aggs1[1], p1a, p1b, cnt0, cnt1,
                        b1.reshape(1, _D), W2)
    aggs2 = _agg_sc(p2a, p2b, srcA, dstA)

    wc2p = jnp.concatenate([Wc2, jnp.zeros((_D, _D - 2), _F32)], axis=1)
    bc2p = jnp.concatenate([bc2, jnp.zeros((_D - 2,), _F32)]).reshape(1, _D)
    outp = _final(h1, aggs2[0], aggs2[1], p2a, p2b, cnt0, cnt1,
                  b2.reshape(1, _D), batch.reshape(_N, 1), Wc1,
                  bc1.reshape(1, _D), wc2p, bc2p)
    return outp[:, :2]


# speedup vs baseline: 2.1593x; 2.1593x over previous
import functools

import jax
import jax.numpy as jnp
from jax import lax
from jax.experimental import pallas as pl
from jax.experimental.pallas import tpu as pltpu
from jax.experimental.pallas import tpu_sc as plsc

_N = 10000
_E = 640000
_D = 128
_B = 64
_BERT = 768
_NC = 2
_NT = 16
_DH = _D // _NC
_NPAD = 10240
_RPT = _NPAD // _NT
_EPW = _E // (_NC * _NT)
_CH = 80
_NCH = _EPW // _CH
_NHALF = _NPAD // _NC
_RPTH = _NHALF // _NT
_ECH = _E // (_NT * _CH)
_GRP = 50
_DGRP = _NCH // _GRP
_CHA = 80
_EPAD = _E
_ACH = _EPAD // (_NT * _CHA)
_AGRP2 = 10
_GRPA = _ACH // _AGRP2
_WB = 64
_DWB = 2048
_BLK = 1000
_F32 = jnp.float32


def _gelu(x):
    return 0.5 * x * (1.0 + lax.erf(x * 0.7071067811865476))


def _embed_body(sem_ref, sidx_ref, wsem_ref, bsem_ref, emb_ref, gamma_ref,
                beta_ref, o_ref):
    h = jnp.dot(sem_ref[...], wsem_ref[...], preferred_element_type=_F32)
    h = h + bsem_ref[...]
    si = sidx_ref[...].astype(jnp.int32)
    h = h + jnp.where(si <= 0, emb_ref[0:1, :], emb_ref[1:2, :])
    m = jnp.mean(h, axis=-1, keepdims=True)
    v = jnp.mean((h - m) * (h - m), axis=-1, keepdims=True)
    h = (h - m) * lax.rsqrt(v + 1e-5) * gamma_ref[...] + beta_ref[...]
    o_ref[...] = _gelu(h)


def _embed(sem, sidx, w_sem, b_sem, emb, gamma, beta):
    grid = (_N // _BLK,)
    return pl.pallas_call(
        _embed_body,
        grid=grid,
        in_specs=[
            pl.BlockSpec((_BLK, _BERT), lambda i: (i, 0)),
            pl.BlockSpec((_BLK, 1), lambda i: (i, 0)),
            pl.BlockSpec((_BERT, _D), lambda i: (0, 0)),
            pl.BlockSpec((1, _D), lambda i: (0, 0)),
            pl.BlockSpec((2, _D), lambda i: (0, 0)),
            pl.BlockSpec((1, _D), lambda i: (0, 0)),
            pl.BlockSpec((1, _D), lambda i: (0, 0)),
        ],
        out_specs=pl.BlockSpec((_BLK, _D), lambda i: (i, 0)),
        out_shape=jax.ShapeDtypeStruct((_N, _D), _F32),
    )(sem, sidx, w_sem, b_sem, emb, gamma, beta)


def _prep_body(h_ref, w_ref, cnt0_ref, cnt1_ref, pa_ref, pb_ref):
    g = lax.rsqrt(cnt0_ref[...] + cnt1_ref[...] + 1.0)
    p = jnp.dot(h_ref[...], w_ref[...], preferred_element_type=_F32) * g
    pa_ref[...] = p[:, :_DH]
    pb_ref[...] = p[:, _DH:]


def _prep(h, w, cnt0, cnt1):
    grid = (_N // _BLK,)
    half = pl.BlockSpec((_BLK, _DH), lambda i: (i, 0))
    return pl.pallas_call(
        _prep_body,
        grid=grid,
        in_specs=[
            pl.BlockSpec((_BLK, _D), lambda i: (i, 0)),
            pl.BlockSpec((_D, _D), lambda i: (0, 0)),
            pl.BlockSpec((_BLK, 1), lambda i: (i, 0)),
            pl.BlockSpec((_BLK, 1), lambda i: (i, 0)),
        ],
        out_specs=[half, half],
        out_shape=[
            jax.ShapeDtypeStruct((_NPAD, _DH), _F32),
            jax.ShapeDtypeStruct((_NPAD, _DH), _F32),
        ],
    )(h, w, cnt0, cnt1)


def _mid_body(agga_ref, aggb_ref, pa_ref, pb_ref, cnt0_ref, cnt1_ref,
              b1_ref, w2_ref, h1_ref, p2a_ref, p2b_ref):
    g = lax.rsqrt(cnt0_ref[...] + cnt1_ref[...] + 1.0)
    s = jnp.concatenate(
        [agga_ref[...] + pa_ref[...], aggb_ref[...] + pb_ref[...]], axis=1)
    h1 = _gelu(s * g + b1_ref[...])
    h1_ref[...] = h1
    p2 = jnp.dot(h1, w2_ref[...], preferred_element_type=_F32) * g
    p2a_ref[...] = p2[:, :_DH]
    p2b_ref[...] = p2[:, _DH:]


def _mid(agga, aggb, pa, pb, cnt0, cnt1, b1, w2):
    grid = (_N // _BLK,)
    full = pl.BlockSpec((_BLK, _D), lambda i: (i, 0))
    half = pl.BlockSpec((_BLK, _DH), lambda i: (i, 0))
    one = pl.BlockSpec((_BLK, 1), lambda i: (i, 0))
    return pl.pallas_call(
        _mid_body,
        grid=grid,
        in_specs=[
            half, half, half, half, one, one,
            pl.BlockSpec((1, _D), lambda i: (0, 0)),
            pl.BlockSpec((_D, _D), lambda i: (0, 0)),
        ],
        out_specs=[full, half, half],
        out_shape=[
            jax.ShapeDtypeStruct((_N, _D), _F32),
            jax.ShapeDtypeStruct((_NPAD, _DH), _F32),
            jax.ShapeDtypeStruct((_NPAD, _DH), _F32),
        ],
    )(agga, aggb, pa, pb, cnt0, cnt1, b1, w2)


def _final_body(h1_ref, agga_ref, aggb_ref, pa_ref, pb_ref, cnt0_ref,
                cnt1_ref, b2_ref, batch_ref, wc1_ref, bc1_ref, wc2_ref,
                bc2_ref, o_ref, sums_scr, counts_scr):
    i = pl.program_id(0)

    @pl.when(i == 0)
    def _():
        sums_scr[...] = jnp.zeros_like(sums_scr)
        counts_scr[...] = jnp.zeros_like(counts_scr)

    g = lax.rsqrt(cnt0_ref[...] + cnt1_ref[...] + 1.0)
    s = jnp.concatenate(
        [agga_ref[...] + pa_ref[...], aggb_ref[...] + pb_ref[...]], axis=1)
    h2 = h1_ref[...] + _gelu(s * g + b2_ref[...])
    onehot = (batch_ref[...] ==
              lax.broadcasted_iota(jnp.int32, (_BLK, _B), 1)).astype(_F32)
    dn = (((0,), (0,)), ((), ()))
    sums_scr[...] += lax.dot_general(onehot, h2, dn,
                                     preferred_element_type=_F32)
    counts_scr[...] += lax.dot_general(onehot, jnp.ones((_BLK, 1), _F32), dn,
                                       preferred_element_type=_F32)

    @pl.when(i == _N // _BLK - 1)
    def _():
        hg = sums_scr[...] / jnp.maximum(counts_scr[...], 1.0)
        z = _gelu(jnp.dot(hg, wc1_ref[...], preferred_element_type=_F32)
                  + bc1_ref[...])
        o_ref[...] = (jnp.dot(z, wc2_ref[...], preferred_element_type=_F32)
                      + bc2_ref[...])


def _final(h1, agga, aggb, pa, pb, cnt0, cnt1, b2, batch, wc1, bc1, wc2p,
           bc2p):
    grid = (_N // _BLK,)
    full = pl.BlockSpec((_BLK, _D), lambda i: (i, 0))
    half = pl.BlockSpec((_BLK, _DH), lambda i: (i, 0))
    one = pl.BlockSpec((_BLK, 1), lambda i: (i, 0))
    wfull = pl.BlockSpec((_D, _D), lambda i: (0, 0))
    brow = pl.BlockSpec((1, _D), lambda i: (0, 0))
    return pl.pallas_call(
        _final_body,
        grid=grid,
        in_specs=[full, half, half, half, half, one, one, brow,
                  pl.BlockSpec((_BLK, 1), lambda i: (i, 0)),
                  wfull, brow, wfull, brow],
        out_specs=pl.BlockSpec((_B, _D), lambda i: (0, 0)),
        out_shape=jax.ShapeDtypeStruct((_B, _D), _F32),
        scratch_shapes=[
            pltpu.VMEM((_B, _D), _F32),
            pltpu.VMEM((_B, 1), _F32),
        ],
        compiler_params=pltpu.CompilerParams(
            dimension_semantics=("arbitrary",)),
    )(h1, agga, aggb, pa, pb, cnt0, cnt1, b2, batch, wc1, bc1, wc2p, bc2p)


def _deg_sc(dst4):
    mesh = plsc.VectorSubcoreMesh(core_axis_name="c", subcore_axis_name="s",
                                  num_cores=_NC, num_subcores=_NT)

    @functools.partial(
        pl.kernel,
        out_type=jax.ShapeDtypeStruct((_NC, _NPAD), _F32),
        mesh=mesh,
        scratch_types=[
            pltpu.VMEM((_GRP, _CH), jnp.int32),
            pltpu.VMEM((_CH,), _F32),
            pltpu.VMEM((_DWB,), _F32),
            pltpu.VMEM_SHARED((_NPAD,), _F32),
            pltpu.SemaphoreType.DMA,
        ],
    )
    def k(dst_hbm, cnt_hbm, didx_v, ones_v, buf_v, acc_sh, dsem):
        c = lax.axis_index("c")
        s = lax.axis_index("s")
        wid = c * _NT + s

        for kk in range(_CH // 16):
            ones_v[pl.ds(kk * 16, 16)] = jnp.ones((16,), _F32)

        @pl.when(s == 0)
        def _():
            def fill_zero(i, _):
                buf_v[pl.ds(i * 16, 16)] = jnp.zeros((16,), _F32)
                return 0
            lax.fori_loop(0, _DWB // 16, fill_zero, 0)
            for t in range(_NPAD // _DWB):
                pltpu.sync_copy(buf_v, acc_sh.at[pl.ds(t * _DWB, _DWB)])

        plsc.subcore_barrier()

        def group(gi, _):
            pltpu.sync_copy(dst_hbm.at[wid, gi], didx_v)
            for sb in range(_GRP // 10):
                ds = [pltpu.async_copy(
                    ones_v, acc_sh.at[didx_v.at[sb * 10 + t]], dsem,
                    add=True) for t in range(10)]
                for d in ds:
                    d.wait()
            return 0
        lax.fori_loop(0, _DGRP, group, 0)

        plsc.subcore_barrier()

        @pl.when(s == 0)
        def _():
            for t in range(_NPAD // _DWB):
                pltpu.sync_copy(acc_sh.at[pl.ds(t * _DWB, _DWB)], buf_v)
                pltpu.sync_copy(buf_v, cnt_hbm.at[c, pl.ds(t * _DWB, _DWB)])

    return k(dst4)


def _agg_sc(pa, pb, srcA, dstA):
    mesh = plsc.VectorSubcoreMesh(core_axis_name="c", subcore_axis_name="s",
                                  num_cores=_NC, num_subcores=_NT)

    @functools.partial(
        pl.kernel,
        out_type=jax.ShapeDtypeStruct((_NC, _NPAD, _DH), _F32),
        mesh=mesh,
        scratch_types=[
            pltpu.VMEM((_GRPA, _CHA), jnp.int32),
            pltpu.VMEM((_GRPA, _CHA), jnp.int32),
            pltpu.VMEM((_CHA, _DH), _F32),
            pltpu.VMEM((_CHA, _DH), _F32),
            pltpu.VMEM((_CHA, _DH), _F32),
            pltpu.VMEM((_CHA, _DH), _F32),
            pltpu.VMEM((_CHA, _DH), _F32),
            pltpu.VMEM((_WB, _DH), _F32),
            pltpu.VMEM_SHARED((_NPAD, _DH), _F32),
            pltpu.SemaphoreType.DMA,
            pltpu.SemaphoreType.DMA,
            pltpu.SemaphoreType.DMA,
            pltpu.SemaphoreType.DMA,
            pltpu.SemaphoreType.DMA,
            pltpu.SemaphoreType.DMA,
            pltpu.SemaphoreType.DMA,
            pltpu.SemaphoreType.DMA,
            pltpu.SemaphoreType.DMA,
            pltpu.SemaphoreType.DMA,
        ],
        compiler_params=pltpu.CompilerParams(use_tc_tiling_on_sc=False),
    )
    def k(pa_hbm, pb_hbm, src_hbm, dst_hbm, out_hbm,
          sidx_v, didx_v, r0, r1, r2, r3, r4, buf_v, acc_sh,
          g0, g1, g2, g3, g4, s0, s1, s2, s3, s4):
        c = lax.axis_index("c")
        s = lax.axis_index("s")
        rows = (r0, r1, r2, r3, r4)
        gsem = (g0, g1, g2, g3, g4)
        ssem = (s0, s1, s2, s3, s4)

        def fill_zero(i, _):
            for jj in range(_DH // 16):
                buf_v[i, pl.ds(jj * 16, 16)] = jnp.zeros((16,), _F32)
            return 0
        lax.fori_loop(0, _WB, fill_zero, 0)

        for t in range(_RPT // _WB):
            pltpu.sync_copy(
                buf_v, acc_sh.at[pl.ds(s * _RPT + t * _WB, _WB)])

        plsc.subcore_barrier()

        def gath(j, b):
            @pl.when(c == 0)
            def _():
                pltpu.async_copy(pa_hbm.at[sidx_v.at[j]], rows[b], gsem[b])

            @pl.when(c == 1)
            def _():
                pltpu.async_copy(pb_hbm.at[sidx_v.at[j]], rows[b], gsem[b])

        def gwait(j, b):
            pltpu.make_async_copy(pa_hbm.at[sidx_v.at[j]], rows[b],
                                  gsem[b]).wait()

        def scat(j, b):
            pltpu.async_copy(rows[b], acc_sh.at[didx_v.at[j]],
                             ssem[b], add=True)

        def swait(j, b):
            pltpu.make_async_copy(rows[b], acc_sh.at[didx_v.at[j]],
                                  ssem[b]).wait()

        def group(gi, _):
            pltpu.sync_copy(src_hbm.at[s, gi], sidx_v)
            pltpu.sync_copy(dst_hbm.at[s, gi], didx_v)

            gath(0, 0)
            gath(1, 1)
            gath(2, 2)
            gwait(0, 0)
            scat(0, 0)
            gath(3, 3)
            gwait(1, 1)
            scat(1, 1)
            gath(4, 4)
            gwait(2, 2)
            scat(2, 2)
            swait(0, 0)
            gath(5, 0)
            gwait(3, 3)
            scat(3, 3)
            swait(1, 1)
            gath(6, 1)
            gwait(4, 4)
            scat(4, 4)
            swait(2, 2)
            gath(7, 2)

            def steady(jo, _2):
                j = jo * 5
                for b in range(5):
                    gwait(j + b, b)
                    scat(j + b, b)
                    bn = (b + 3) % 5
                    swait(j + b - 2, bn)
                    gath(j + b + 3, bn)
                return 0
            lax.fori_loop(1, _GRPA // 5 - 1, steady, 0)

            gg = _GRPA - 5
            gwait(gg, 0)
            scat(gg, 0)
            swait(gg - 2, 3)
            gath(gg + 3, 3)
            gwait(gg + 1, 1)
            scat(gg + 1, 1)
            swait(gg - 1, 4)
            gath(gg + 4, 4)
            gwait(gg + 2, 2)
            scat(gg + 2, 2)
            gwait(gg + 3, 3)
            scat(gg + 3, 3)
            gwait(gg + 4, 4)
            scat(gg + 4, 4)
            for j, b in ((gg, 0), (gg + 1, 1), (gg + 2, 2), (gg + 3, 3),
                         (gg + 4, 4)):
                swait(j, b)
            return 0
        lax.fori_loop(0, _AGRP2, group, 0)

        plsc.subcore_barrier()
        for t in range(_RPT // _WB):
            pltpu.sync_copy(
                acc_sh.at[pl.ds(s * _RPT + t * _WB, _WB)], buf_v)
            pltpu.sync_copy(
                buf_v, out_hbm.at[c, pl.ds(s * _RPT + t * _WB, _WB)])

    return k(pa, pb, srcA, dstA)


def kernel(x, edge_index, batch, W_sem, b_sem, emb, gamma, beta, W1, b1, W2,
           b2, Wc1, bc1, Wc2, bc2):
    sem_feat = x[:, :_BERT]
    sidx = x[:, _BERT:]
    src = edge_index[0]
    dst = edge_index[1]
    dst4 = dst.reshape(_NC * _NT, _DGRP, _GRP, _CH)
    if _EPAD > _E:
        epad = jnp.full((_EPAD - _E,), _NPAD - 1, jnp.int32)
        src = jnp.concatenate([src, epad])
        dst = jnp.concatenate([dst, epad])
    srcA = src.reshape(_NT, _AGRP2, _GRPA, _CHA)
    dstA = dst.reshape(_NT, _AGRP2, _GRPA, _CHA)

    cnts = _deg_sc(dst4)
    cnt0 = cnts[0].reshape(_NPAD, 1)
    cnt1 = cnts[1].reshape(_NPAD, 1)

    h0 = _embed(sem_feat, sidx, W_sem, b_sem.reshape(1, _D), emb,
                gamma.reshape(1, _D), beta.reshape(1, _D))

    p1a, p1b = _prep(h0, W1, cnt0, cnt1)
    aggs1 = _agg_sc(p1a, p1b, srcA, dstA)
    h1, p2a, p2b = _mid(aggs1[0], aggs1[1], p1a, p1b, cnt0, cnt1,
                        b1.reshape(1, _D), W2)
    aggs2 = _agg_sc(p2a, p2b, srcA, dstA)

    wc2p = jnp.concatenate([Wc2, jnp.zeros((_D, _D - 2), _F32)], axis=1)
    bc2p = jnp.concatenate([bc2, jnp.zeros((_D - 2,), _F32)]).reshape(1, _D)
    outp = _final(h1, aggs2[0], aggs2[1], p2a, p2b, cnt0, cnt1,
                  b2.reshape(1, _D), batch.reshape(_N, 1), Wc1,
                  bc1.reshape(1, _D), wc2p, bc2p)
    return outp[:, :2]
